# baseline (speedup 1.0000x reference)
import functools

import jax
import jax.numpy as jnp
from jax import lax
from jax.experimental import pallas as pl
from jax.experimental.pallas import tpu as pltpu
from jax.experimental.pallas import tpu_sc as plsc

_L = 16


def _make_sc_kernel(n, d, p):
    nchunk = d // _L
    tg = 19
    t = tg * _L
    nt = n // t
    nw = 32
    rounds = (nt + nw - 1) // nw
    last_w = nt - (rounds - 1) * nw
    nbuf = 3
    pieces = [(0, 128), (128, 128), (256, t - 256)]

    mesh = plsc.VectorSubcoreMesh(core_axis_name="c", subcore_axis_name="s")

    @functools.partial(
        pl.kernel,
        out_type=jax.ShapeDtypeStruct((n * d,), jnp.float32),
        mesh=mesh,
        scratch_types=[
            pltpu.VMEM((p * d,), jnp.float32),
            pltpu.VMEM((p * d,), jnp.float32),
            pltpu.VMEM((p * _L,), jnp.float32),
            pltpu.VMEM((t * d,), jnp.float32),
            pltpu.VMEM((t * d,), jnp.float32),
            pltpu.VMEM((t * d,), jnp.float32),
            pltpu.VMEM((t,), jnp.float32),
            pltpu.VMEM((t,), jnp.float32),
            pltpu.VMEM((t,), jnp.float32),
            pltpu.VMEM((t,), jnp.int32),
            pltpu.VMEM((t,), jnp.int32),
            pltpu.VMEM((t,), jnp.int32),
            pltpu.SemaphoreType.DMA,
            pltpu.SemaphoreType.DMA,
            pltpu.SemaphoreType.DMA,
            pltpu.SemaphoreType.DMA,
            pltpu.SemaphoreType.DMA,
            pltpu.SemaphoreType.DMA,
            pltpu.SemaphoreType.DMA,
            pltpu.SemaphoreType.DMA,
            pltpu.SemaphoreType.DMA,
        ],
    )
    def sc_k(x_hbm, lup_hbm, lp_hbm, bcol_hbm, out_hbm,
             lup_v, lp_v, bcol_v, buf0, buf1, buf2, col0, col1, col2,
             cix0, cix1, cix2,
             isem0, isem1, isem2, osem0, osem1, osem2, csem0, csem1, csem2):
        bufs = (buf0, buf1, buf2)
        cols = (col0, col1, col2)
        cixs = (cix0, cix1, cix2)
        isems = (isem0, isem1, isem2)
        osems = (osem0, osem1, osem2)
        csems = (csem0, csem1, csem2)
        wid = lax.axis_index("c") * 16 + lax.axis_index("s")

        pltpu.sync_copy(lup_hbm, lup_v)
        pltpu.sync_copy(lp_hbm, lp_v)
        pltpu.sync_copy(bcol_hbm, bcol_v)

        iota = lax.iota(jnp.int32, _L)
        zeros = jnp.zeros((_L,), jnp.int32)

        def base0(k):
            return (k * nw + wid) * (t * d)

        def in_copy(k, b):
            return pltpu.make_async_copy(
                x_hbm.at[pl.ds(base0(k), t * d)], bufs[b], isems[b])

        def out_copy(k, b):
            return pltpu.make_async_copy(
                bufs[b], out_hbm.at[pl.ds(base0(k), t * d)], osems[b])

        def col_start(k, b):
            bvec = (zeros + base0(k)) + iota * d
            for g in range(tg):
                cixs[b][pl.ds(g * _L, _L)] = bvec + (g * _L * d)
            for off, sz in pieces:
                pltpu.make_async_copy(
                    x_hbm.at[cixs[b].at[pl.ds(off, sz)]],
                    cols[b].at[pl.ds(off, sz)], csems[b]).start()

        def col_wait(b):
            pltpu.make_async_copy(
                x_hbm.at[cixs[b]], cols[b], csems[b]).wait()

        def compute(buf, col):
            def gbody(g, carry):
                onesf = jnp.full((_L,), 1.0, jnp.float32)
                zerosf = jnp.full((_L,), 0.0, jnp.float32)
                v = col[pl.ds(g * _L, _L)]
                zacc = zerosf
                for pp in range(p):
                    u = bcol_v[pl.ds(pp * _L, _L)]
                    zacc = zacc + jnp.where(v == u, onesf, zerosf)
                s = zacc[0]
                for i in range(1, _L):
                    s = s + zacc[i]
                anyhit = s > 0.0

                @pl.when(anyhit)
                def _rare():
                    def rbody(r, rcarry):
                        rbase = (g * _L + r) * d

                        def qbody(q, qcarry):
                            ones2 = jnp.full((_L,), 1.0, jnp.float32)
                            zeros2 = jnp.full((_L,), 0.0, jnp.float32)
                            ch0 = buf[pl.ds(rbase, _L)]
                            lq0 = lup_v[pl.ds(q * d, _L)]
                            e0 = jnp.where(ch0 == lq0, ones2, zeros2)
                            sc = e0[0]
                            for i in range(1, _L):
                                sc = sc * e0[i]
                            cand = sc > 0.0

                            @pl.when(cand)
                            def _verify():
                                ones3 = jnp.full((_L,), 1.0, jnp.float32)
                                zeros3 = jnp.full((_L,), 0.0, jnp.float32)
                                pacc = ones3
                                for c in range(nchunk):
                                    lqc = lup_v[pl.ds(q * d + c * _L, _L)]
                                    chc = buf[pl.ds(rbase + c * _L, _L)]
                                    pacc = pacc * jnp.where(
                                        chc == lqc, ones3, zeros3)
                                sm = pacc[0]
                                for i in range(1, _L):
                                    sm = sm * pacc[i]

                                @pl.when(sm > 0.0)
                                def _overwrite():
                                    for c in range(nchunk):
                                        buf[pl.ds(rbase + c * _L, _L)] = (
                                            lp_v[pl.ds(q * d + c * _L, _L)])

                            return qcarry

                        lax.fori_loop(0, p, qbody, 0)
                        return rcarry

                    lax.fori_loop(0, _L, rbody, 0)

                return carry

            lax.fori_loop(0, tg, gbody, 0)

        for k in range(nbuf):
            in_copy(k, k % nbuf).start()
            col_start(k, k % nbuf)

        for k in range(rounds):
            b = k % nbuf
            if k < rounds - 1:
                in_copy(k, b).wait()
                col_wait(b)
                compute(bufs[b], cols[b])
                out_copy(k, b).start()
            else:
                @pl.when(wid < last_w)
                def _last(k=k, b=b):
                    in_copy(k, b).wait()
                    col_wait(b)
                    compute(bufs[b], cols[b])
                    out_copy(k, b).start()
            if k + nbuf <= rounds - 1:
                out_copy(k, b).wait()
                if k + nbuf < rounds - 1:
                    in_copy(k + nbuf, b).start()
                    col_start(k + nbuf, b)
                else:
                    @pl.when(wid < last_w)
                    def _pf(k=k, b=b):
                        in_copy(k + nbuf, b).start()
                        col_start(k + nbuf, b)

        for k in range(max(0, rounds - nbuf), rounds):
            b = k % nbuf
            if k < rounds - 1:
                out_copy(k, b).wait()
            else:
                @pl.when(wid < last_w)
                def _wlast(k=k, b=b):
                    out_copy(k, b).wait()

    return sc_k


@jax.jit
def kernel(x, batch, learnable_param, last_updated_param):
    del batch
    n, d = x.shape
    p = last_updated_param.shape[0]
    bcol = jnp.repeat(last_updated_param[:, 0:1], _L, axis=1)
    sc_k = _make_sc_kernel(n, d, p)
    out = sc_k(x.reshape(-1), last_updated_param.reshape(-1),
               learnable_param.reshape(-1), bcol.reshape(-1))
    return out.reshape(n, d)


# speedup vs baseline: 1.0010x; 1.0010x over previous
import functools

import jax
import jax.numpy as jnp
from jax import lax
from jax.experimental import pallas as pl
from jax.experimental.pallas import tpu as pltpu
from jax.experimental.pallas import tpu_sc as plsc

_L = 16


def _piece_list(nrows):
    out, off = [], 0
    while off < nrows:
        sz = min(128, nrows - off)
        out.append((off, sz))
        off += sz
    return out


def _make_sc_kernel(n, d, p):
    nchunk = d // _L
    tg = 14
    t = tg * _L
    ngrp = n // _L
    nt_full = ngrp // tg
    rem_g = ngrp - nt_full * tg
    tp = rem_g * _L
    nt = nt_full + (1 if rem_g else 0)
    nw = 32
    rounds = (nt + nw - 1) // nw
    fw = nt_full - (rounds - 1) * nw
    nbuf = 4
    look = 2

    mesh = plsc.VectorSubcoreMesh(core_axis_name="c", subcore_axis_name="s")

    @functools.partial(
        pl.kernel,
        out_type=jax.ShapeDtypeStruct((n * d,), jnp.float32),
        mesh=mesh,
        scratch_types=[
            pltpu.VMEM((p * d,), jnp.float32),
            pltpu.VMEM((p * d,), jnp.float32),
            pltpu.VMEM((p * _L,), jnp.float32),
            pltpu.VMEM((t * d,), jnp.float32),
            pltpu.VMEM((t * d,), jnp.float32),
            pltpu.VMEM((t * d,), jnp.float32),
            pltpu.VMEM((t * d,), jnp.float32),
            pltpu.VMEM((t,), jnp.float32),
            pltpu.VMEM((t,), jnp.float32),
            pltpu.VMEM((t,), jnp.float32),
            pltpu.VMEM((t,), jnp.float32),
            pltpu.VMEM((t,), jnp.int32),
            pltpu.VMEM((t,), jnp.int32),
            pltpu.VMEM((t,), jnp.int32),
            pltpu.VMEM((t,), jnp.int32),
            pltpu.SemaphoreType.DMA,
            pltpu.SemaphoreType.DMA,
            pltpu.SemaphoreType.DMA,
            pltpu.SemaphoreType.DMA,
            pltpu.SemaphoreType.DMA,
            pltpu.SemaphoreType.DMA,
            pltpu.SemaphoreType.DMA,
            pltpu.SemaphoreType.DMA,
            pltpu.SemaphoreType.DMA,
            pltpu.SemaphoreType.DMA,
            pltpu.SemaphoreType.DMA,
            pltpu.SemaphoreType.DMA,
        ],
    )
    def sc_k(x_hbm, lup_hbm, lp_hbm, bcol_hbm, out_hbm,
             lup_v, lp_v, bcol_v,
             buf0, buf1, buf2, buf3, col0, col1, col2, col3,
             cix0, cix1, cix2, cix3,
             isem0, isem1, isem2, isem3,
             osem0, osem1, osem2, osem3,
             csem0, csem1, csem2, csem3):
        bufs = (buf0, buf1, buf2, buf3)
        cols = (col0, col1, col2, col3)
        cixs = (cix0, cix1, cix2, cix3)
        isems = (isem0, isem1, isem2, isem3)
        osems = (osem0, osem1, osem2, osem3)
        csems = (csem0, csem1, csem2, csem3)
        wid = lax.axis_index("c") * 16 + lax.axis_index("s")

        pltpu.sync_copy(lup_hbm, lup_v)
        pltpu.sync_copy(lp_hbm, lp_v)
        pltpu.sync_copy(bcol_hbm, bcol_v)

        iota = lax.iota(jnp.int32, _L)
        zeros = jnp.zeros((_L,), jnp.int32)

        def base0(k):
            return (k * nw + wid) * (t * d)

        def in_copy(k, b, rows):
            return pltpu.make_async_copy(
                x_hbm.at[pl.ds(base0(k), rows * d)],
                bufs[b].at[pl.ds(0, rows * d)], isems[b])

        def out_copy(k, b, rows):
            return pltpu.make_async_copy(
                bufs[b].at[pl.ds(0, rows * d)],
                out_hbm.at[pl.ds(base0(k), rows * d)], osems[b])

        def col_start(k, b, gg):
            bvec = (zeros + base0(k)) + iota * d
            for g in range(gg):
                cixs[b][pl.ds(g * _L, _L)] = bvec + (g * _L * d)
            for off, sz in _piece_list(gg * _L):
                pltpu.make_async_copy(
                    x_hbm.at[cixs[b].at[pl.ds(off, sz)]],
                    cols[b].at[pl.ds(off, sz)], csems[b]).start()

        def col_wait(b, gg):
            pltpu.make_async_copy(
                x_hbm.at[cixs[b].at[pl.ds(0, gg * _L)]],
                cols[b].at[pl.ds(0, gg * _L)], csems[b]).wait()

        def compute(buf, col, gg):
            def gbody(g, carry):
                onesf = jnp.full((_L,), 1.0, jnp.float32)
                zerosf = jnp.full((_L,), 0.0, jnp.float32)
                v = col[pl.ds(g * _L, _L)]
                zacc = zerosf
                for pp in range(p):
                    u = bcol_v[pl.ds(pp * _L, _L)]
                    zacc = zacc + jnp.where(v == u, onesf, zerosf)
                s = zacc[0]
                for i in range(1, _L):
                    s = s + zacc[i]
                anyhit = s > 0.0

                @pl.when(anyhit)
                def _rare():
                    def rbody(r, rcarry):
                        rbase = (g * _L + r) * d

                        def qbody(q, qcarry):
                            ones2 = jnp.full((_L,), 1.0, jnp.float32)
                            zeros2 = jnp.full((_L,), 0.0, jnp.float32)
                            ch0 = buf[pl.ds(rbase, _L)]
                            lq0 = lup_v[pl.ds(q * d, _L)]
                            e0 = jnp.where(ch0 == lq0, ones2, zeros2)
                            sc = e0[0]
                            for i in range(1, _L):
                                sc = sc * e0[i]
                            cand = sc > 0.0

                            @pl.when(cand)
                            def _verify():
                                ones3 = jnp.full((_L,), 1.0, jnp.float32)
                                zeros3 = jnp.full((_L,), 0.0, jnp.float32)
                                pacc = ones3
                                for c in range(nchunk):
                                    lqc = lup_v[pl.ds(q * d + c * _L, _L)]
                                    chc = buf[pl.ds(rbase + c * _L, _L)]
                                    pacc = pacc * jnp.where(
                                        chc == lqc, ones3, zeros3)
                                sm = pacc[0]
                                for i in range(1, _L):
                                    sm = sm * pacc[i]

                                @pl.when(sm > 0.0)
                                def _overwrite():
                                    for c in range(nchunk):
                                        buf[pl.ds(rbase + c * _L, _L)] = (
                                            lp_v[pl.ds(q * d + c * _L, _L)])

                            return qcarry

                        lax.fori_loop(0, p, qbody, 0)
                        return rcarry

                    lax.fori_loop(0, _L, rbody, 0)

                return carry

            lax.fori_loop(0, gg, gbody, 0)

        def launch_in(j):
            b = j % nbuf
            if j < rounds - 1:
                in_copy(j, b, t).start()
                col_start(j, b, tg)
            else:
                @pl.when(wid < fw)
                def _f(j=j, b=b):
                    in_copy(j, b, t).start()
                    col_start(j, b, tg)
                if rem_g:
                    @pl.when(wid == fw)
                    def _p(j=j, b=b):
                        in_copy(j, b, tp).start()
                        col_start(j, b, rem_g)

        def run_round(k):
            b = k % nbuf
            if k < rounds - 1:
                in_copy(k, b, t).wait()
                col_wait(b, tg)
                compute(bufs[b], cols[b], tg)
                out_copy(k, b, t).start()
            else:
                @pl.when(wid < fw)
                def _f(k=k, b=b):
                    in_copy(k, b, t).wait()
                    col_wait(b, tg)
                    compute(bufs[b], cols[b], tg)
                    out_copy(k, b, t).start()
                if rem_g:
                    @pl.when(wid == fw)
                    def _p(k=k, b=b):
                        in_copy(k, b, tp).wait()
                        col_wait(b, rem_g)
                        compute(bufs[b], cols[b], rem_g)
                        out_copy(k, b, tp).start()

        def drain_out(j):
            b = j % nbuf
            if j < rounds - 1:
                out_copy(j, b, t).wait()
            else:
                @pl.when(wid < fw)
                def _f(j=j, b=b):
                    out_copy(j, b, t).wait()
                if rem_g:
                    @pl.when(wid == fw)
                    def _p(j=j, b=b):
                        out_copy(j, b, tp).wait()

        for j in range(min(look, rounds)):
            launch_in(j)

        drained = 0
        for k in range(rounds):
            run_round(k)
            j = k + look
            if j <= rounds - 1:
                if k - look >= 0:
                    drain_out(k - look)
                    drained = k - look + 1
                launch_in(j)

        for j in range(drained, rounds):
            drain_out(j)

    return sc_k


@jax.jit
def kernel(x, batch, learnable_param, last_updated_param):
    del batch
    n, d = x.shape
    p = last_updated_param.shape[0]
    bcol = jnp.repeat(last_updated_param[:, 0:1], _L, axis=1)
    sc_k = _make_sc_kernel(n, d, p)
    out = sc_k(x.reshape(-1), last_updated_param.reshape(-1),
               learnable_param.reshape(-1), bcol.reshape(-1))
    return out.reshape(n, d)
